# Initial kernel scaffold; baseline (speedup 1.0000x reference)
import jax
import jax.numpy as jnp
from jax.experimental import pallas as pl


def kernel(feature, edge_index, user, item_p, item_n, cat_p, cat_n, price_p, price_n, W, b):
    raise NotImplementedError("write your pallas kernel here")


# speedup vs baseline: 4.7780x; 4.7780x over previous
import functools

import jax
import jax.numpy as jnp
from jax import lax
from jax.experimental import pallas as pl
from jax.experimental.pallas import tpu as pltpu
import jax.experimental.pallas.tpu_sc as plsc

SPLIT = 64
ALPHA = 0.5

_NC = 2
_NS = 16
_NW = _NC * _NS

_mesh = plsc.VectorSubcoreMesh(core_axis_name="c", subcore_axis_name="s")


def _mm_body(f_ref, w_ref, o_ref):
    o_ref[...] = jnp.dot(f_ref[...], w_ref[...],
                         preferred_element_type=jnp.float32)


def _matmul(feature, W):
    N, D = feature.shape
    return pl.pallas_call(
        _mm_body,
        out_shape=jax.ShapeDtypeStruct((N, D), jnp.float32),
    )(feature, W)


def _segsum_sc(xw, src, dst):
    N, D = xw.shape
    E = src.shape[0]
    EPW = E // _NW
    CH = 128
    n_full = EPW // CH
    rem = EPW - n_full * CH
    assert n_full * CH + rem == EPW and EPW * _NW == E
    NPT = (N // _NS) // 8 * 8
    TAIL = N - NPT * _NS
    assert TAIL % 8 == 0 and TAIL <= CH

    @functools.partial(
        pl.kernel,
        out_type=jax.ShapeDtypeStruct((_NC, N, D), jnp.float32),
        mesh=_mesh,
        scratch_types=[
            pltpu.VMEM((CH,), jnp.int32),
            pltpu.VMEM((CH,), jnp.int32),
            pltpu.VMEM((CH, D), jnp.float32),
            pltpu.VMEM_SHARED((N, D), jnp.float32),
            pltpu.SemaphoreType.DMA,
        ],
    )
    def k(xw_hbm, src_hbm, dst_hbm, out_hbm, sidx, didx, rows, acc, sem):
        c = lax.axis_index("c")
        s = lax.axis_index("s")
        wid = c * _NS + s

        def zrow(i, _):
            for j in range(D // 16):
                rows[i, pl.ds(j * 16, 16)] = jnp.zeros((16,), jnp.float32)
            return 0
        lax.fori_loop(0, CH, zrow, 0)

        zbase = pl.multiple_of(s * NPT, 8)
        n_zfull = NPT // CH
        zrem = NPT - n_zfull * CH
        for t in range(n_zfull):
            pltpu.sync_copy(rows, acc.at[pl.ds(zbase + t * CH, CH)])
        if zrem:
            pltpu.sync_copy(rows.at[pl.ds(0, zrem)],
                            acc.at[pl.ds(zbase + n_zfull * CH, zrem)])

        @pl.when(s == _NS - 1)
        def _():
            pltpu.sync_copy(rows.at[pl.ds(0, TAIL)],
                            acc.at[pl.ds(NPT * _NS, TAIL)])
        plsc.subcore_barrier()

        base = pl.multiple_of(wid * EPW, 8)

        def body(i, _):
            off = pl.multiple_of(base + i * CH, 8)
            pltpu.sync_copy(src_hbm.at[pl.ds(off, CH)], sidx)
            pltpu.sync_copy(dst_hbm.at[pl.ds(off, CH)], didx)
            pltpu.async_copy(xw_hbm.at[sidx], rows, sem).wait()
            pltpu.sync_copy(rows, acc.at[didx], add=True)
            return 0
        lax.fori_loop(0, n_full, body, 0)

        if rem:
            off = pl.multiple_of(base + n_full * CH, 8)
            pltpu.sync_copy(src_hbm.at[pl.ds(off, rem)],
                            sidx.at[pl.ds(0, rem)])
            pltpu.sync_copy(dst_hbm.at[pl.ds(off, rem)],
                            didx.at[pl.ds(0, rem)])
            pltpu.async_copy(xw_hbm.at[sidx.at[pl.ds(0, rem)]],
                             rows.at[pl.ds(0, rem)], sem).wait()
            pltpu.sync_copy(rows.at[pl.ds(0, rem)],
                            acc.at[didx.at[pl.ds(0, rem)]], add=True)
        plsc.subcore_barrier()

        for t in range(n_zfull):
            pltpu.sync_copy(acc.at[pl.ds(zbase + t * CH, CH)],
                            out_hbm.at[c, pl.ds(zbase + t * CH, CH)])
        if zrem:
            pltpu.sync_copy(acc.at[pl.ds(zbase + n_zfull * CH, zrem)],
                            out_hbm.at[c, pl.ds(zbase + n_zfull * CH, zrem)])

        @pl.when(s == _NS - 1)
        def _():
            pltpu.sync_copy(acc.at[pl.ds(NPT * _NS, TAIL)],
                            out_hbm.at[c, pl.ds(NPT * _NS, TAIL)])

    return k(xw, src, dst)


def _comb_body(a_ref, b_ref, o_ref):
    o_ref[...] = jnp.tanh(a_ref[0] + a_ref[1] + b_ref[...])


def _combine(partials, b):
    NC, N, D = partials.shape
    return pl.pallas_call(
        _comb_body,
        out_shape=jax.ShapeDtypeStruct((N, D), jnp.float32),
    )(partials, b)


def _gather_sc(x, idxs):
    N, D = x.shape
    B = idxs[0].shape[0]
    BPW = B // _NW
    CH = 128
    n_ch = BPW // CH
    assert n_ch * CH == BPW
    n_arr = len(idxs)

    @functools.partial(
        pl.kernel,
        out_type=[jax.ShapeDtypeStruct((B, D), jnp.float32)
                  for _ in range(n_arr)],
        mesh=_mesh,
        scratch_types=[
            pltpu.VMEM((CH,), jnp.int32),
            pltpu.VMEM((CH, D), jnp.float32),
            pltpu.SemaphoreType.DMA,
        ],
    )
    def k(x_hbm, *rest):
        idx_hbms = rest[:n_arr]
        out_hbms = rest[n_arr:2 * n_arr]
        iv, rows, sem = rest[2 * n_arr:]
        c = lax.axis_index("c")
        s = lax.axis_index("s")
        wid = c * _NS + s
        base = pl.multiple_of(wid * BPW, 8)
        for a in range(n_arr):
            for t in range(n_ch):
                off = pl.multiple_of(base + t * CH, 8)
                pltpu.sync_copy(idx_hbms[a].at[pl.ds(off, CH)], iv)
                pltpu.async_copy(x_hbm.at[iv], rows, sem).wait()
                pltpu.sync_copy(rows, out_hbms[a].at[pl.ds(off, CH)])

    return k(x, *idxs)


def _fm_body(u_ref, ip_ref, cp_ref, pp_ref, in_ref, cn_ref, pn_ref,
             op_ref, on_ref):
    u = u_ref[...]
    ug, uc = u[:, :SPLIT], u[:, SPLIT:]

    def fm3(a, b, c):
        sm = a + b + c
        sq = a * a + b * b + c * c
        return jnp.sum(0.5 * (sm * sm - sq), axis=1)

    ip = ip_ref[...]
    cp = cp_ref[...]
    pp = pp_ref[...]
    op_ref[...] = (fm3(ug, ip[:, :SPLIT], pp[:, :SPLIT])
                   + ALPHA * fm3(uc, cp[:, SPLIT:], pp[:, SPLIT:]))
    i_n = in_ref[...]
    c_n = cn_ref[...]
    p_n = pn_ref[...]
    on_ref[...] = (fm3(ug, i_n[:, :SPLIT], p_n[:, :SPLIT])
                   + ALPHA * fm3(uc, c_n[:, SPLIT:], p_n[:, SPLIT:]))


def _fm_decode(u, ip, cp, pp, i_n, c_n, p_n):
    B, D = u.shape
    BLK = 2048
    grid = (B // BLK,)
    mat_spec = pl.BlockSpec((BLK, D), lambda i: (i, 0))
    vec_spec = pl.BlockSpec((BLK,), lambda i: (i,))
    return pl.pallas_call(
        _fm_body,
        grid=grid,
        in_specs=[mat_spec] * 7,
        out_specs=[vec_spec, vec_spec],
        out_shape=[jax.ShapeDtypeStruct((B,), jnp.float32)] * 2,
    )(u, ip, cp, pp, i_n, c_n, p_n)


def kernel(feature, edge_index, user, item_p, item_n, cat_p, cat_n,
           price_p, price_n, W, b):
    src = edge_index[0]
    dst = edge_index[1]
    xw = _matmul(feature, W)
    partials = _segsum_sc(xw, src, dst)
    x = _combine(partials, b)
    u, ip, cp, pp, i_n, c_n, p_n = _gather_sc(
        x, [user, item_p, cat_p, price_p, item_n, cat_n, price_n])
    pred_p, pred_n = _fm_decode(u, ip, cp, pp, i_n, c_n, p_n)
    return (pred_p, pred_n)
